# baseline (speedup 1.0000x reference)
import functools

import jax
import jax.numpy as jnp
from jax import lax
from jax.experimental import pallas as pl
from jax.experimental.pallas import tpu as pltpu
from jax.experimental.pallas import tpu_sc as plsc

_LANE = 128
_FIRE = 16
_FP = 32


@functools.lru_cache(maxsize=None)
def _build_gather(num_idx, d):
    info = plsc.get_sparse_core_info()
    nc, ns = info.num_cores, info.num_subcores
    nw = nc * ns
    per_w = num_idx // nw
    assert per_w * nw == num_idx
    n_stream = per_w // _LANE
    assert n_stream * _LANE == per_w
    n_super = n_stream // _FIRE
    assert n_super * _FIRE == n_stream
    chunk = _FIRE * _LANE
    chunk_prows = chunk * d // 128
    prows_w = per_w * d // 128

    mesh = plsc.VectorSubcoreMesh(core_axis_name="c", subcore_axis_name="s")

    @functools.partial(
        pl.kernel,
        mesh=mesh,
        compiler_params=pltpu.CompilerParams(use_tc_tiling_on_sc=False),
        out_type=jax.ShapeDtypeStruct((num_idx, d), jnp.float32),
        scratch_types=[
            pltpu.VMEM((n_stream, _LANE), jnp.int32),
            pltpu.VMEM((chunk, d), jnp.float32),
            pltpu.SemaphoreType.DMA,
        ],
    )
    def gather_k(idx_hbm, table_hbm, out_hbm, idx_v, rows_v, sem):
        wid = lax.axis_index("s") * nc + lax.axis_index("c")
        pltpu.sync_copy(idx_hbm.at[wid], idx_v)

        def superchunk(s, carry):
            for j in range(_FIRE):
                pltpu.make_async_copy(
                    table_hbm.at[idx_v.at[s * _FIRE + j]],
                    rows_v.at[pl.ds(j * _LANE, _LANE)],
                    sem,
                ).start()
            for j in range(_FIRE):
                pltpu.make_async_copy(
                    table_hbm.at[idx_v.at[s * _FIRE + j]],
                    rows_v.at[pl.ds(j * _LANE, _LANE)],
                    sem,
                ).wait()
            pltpu.sync_copy(
                rows_v,
                out_hbm.at[pl.ds(wid * per_w + s * chunk, chunk)],
            )
            return carry

        lax.fori_loop(0, n_super, superchunk, 0)

    return gather_k, nw, n_stream


@functools.lru_cache(maxsize=None)
def _build_matmul(bsz, d, blk, nj):
    def mm_k(x_ref, w_ref, b_ref, o_ref):
        acc = jnp.broadcast_to(b_ref[...], (blk, d))
        for j in range(nj):
            acc = acc + jnp.dot(
                x_ref[pl.ds(j * blk * d * 8 // 128, blk * d * 8 // 128), :],
                w_ref[pl.ds(j * 128, 128), :],
                preferred_element_type=jnp.float32,
            )
        o_ref[...] = acc

    prows_blk = blk * _FP * d // 128
    return pl.pallas_call(
        mm_k,
        grid=(bsz // blk,),
        in_specs=[
            pl.BlockSpec((prows_blk, 128), lambda i: (i, 0)),
            pl.BlockSpec((nj * 128, d), lambda i: (0, 0)),
            pl.BlockSpec((1, d), lambda i: (0, 0)),
        ],
        out_specs=pl.BlockSpec((blk, d), lambda i: (i, 0)),
        out_shape=jax.ShapeDtypeStruct((bsz, d), jnp.float32),
    )


def kernel(inputs, table, W, b):
    bsz, f = inputs.shape
    d = table.shape[1]
    blk = 512
    nj = _FP // 8
    num_idx = bsz * _FP

    idxp = jnp.concatenate(
        [inputs.astype(jnp.int32),
         jnp.zeros((bsz, _FP - f), jnp.int32)], axis=1)
    idxr = idxp.reshape(bsz // blk, blk, nj, 8).transpose(0, 2, 1, 3)

    gather_fn, nw, n_stream = _build_gather(num_idx, d)
    idx3 = idxr.reshape(nw, n_stream, _LANE)
    x = gather_fn(idx3, table)
    x = x.reshape(num_idx * d // 128, 128)

    wp = jnp.concatenate(
        [W, jnp.zeros(((_FP - f) * d, d), jnp.float32)], axis=0)
    mm = _build_matmul(bsz, d, blk, nj)
    return mm(x, wp, b.reshape(1, d))


# speedup vs baseline: 1.0030x; 1.0030x over previous
import functools

import jax
import jax.numpy as jnp
from jax import lax
from jax.experimental import pallas as pl
from jax.experimental.pallas import tpu as pltpu
from jax.experimental.pallas import tpu_sc as plsc

_LANE = 128
_FIRE = 16
_FP = 32


@functools.lru_cache(maxsize=None)
def _build_gather(num_idx, d):
    info = plsc.get_sparse_core_info()
    nc, ns = info.num_cores, info.num_subcores
    nw = nc * ns
    per_w = num_idx // nw
    assert per_w * nw == num_idx
    n_stream = per_w // _LANE
    assert n_stream * _LANE == per_w
    spl = 128 // d
    n_slab = n_stream // spl
    assert n_slab * spl == n_stream

    mesh = plsc.VectorSubcoreMesh(core_axis_name="c", subcore_axis_name="s")

    nbuf = 3

    @functools.partial(
        pl.kernel,
        mesh=mesh,
        compiler_params=pltpu.CompilerParams(use_tc_tiling_on_sc=False),
        out_type=jax.ShapeDtypeStruct((num_idx * d // 128, 128), jnp.float32),
        scratch_types=(
            [pltpu.VMEM((n_stream, _LANE), jnp.int32)]
            + [pltpu.VMEM((spl * _LANE, d), jnp.float32)] * nbuf
            + [pltpu.SemaphoreType.DMA] * (2 * nbuf)
        ),
    )
    def gather_k(idx_hbm, table_hbm, out_hbm, idx_v, *bufs_sems):
        gbufs = bufs_sems[:nbuf]
        gsems = bufs_sems[nbuf:2 * nbuf]
        osems = bufs_sems[2 * nbuf:]
        wid = lax.axis_index("s") * nc + lax.axis_index("c")
        pltpu.sync_copy(idx_hbm.at[wid], idx_v)

        def stream_copy(t, a):
            return pltpu.make_async_copy(
                table_hbm.at[idx_v.at[t * spl + a]],
                gbufs[t % nbuf].at[pl.ds(a * _LANE, _LANE)],
                gsems[t % nbuf],
            )

        def out_copy(t, a):
            row0 = (wid * n_slab + t) * _LANE
            return pltpu.make_async_copy(
                gbufs[t % nbuf].at[pl.ds(a * _LANE, _LANE)],
                out_hbm.at[pl.ds(row0, _LANE), pl.ds(a * d, d)],
                osems[t % nbuf],
            )

        def fire(t):
            for a in range(spl):
                stream_copy(t, a).start()

        fire(0)
        fire(1)
        for t in range(n_slab):
            for a in range(spl):
                stream_copy(t, a).wait()
            for a in range(spl):
                out_copy(t, a).start()
            if t + 2 < n_slab:
                if t >= 1:
                    for a in range(spl):
                        out_copy(t - 1, a).wait()
                fire(t + 2)
        for t in (n_slab - 2, n_slab - 1):
            for a in range(spl):
                out_copy(t, a).wait()

    return gather_k, nw, n_stream


@functools.lru_cache(maxsize=None)
def _build_matmul(bsz, d, blk, nj):
    def mm_k(x_ref, w_ref, b_ref, o_ref):
        acc = jnp.broadcast_to(b_ref[...], (blk, d))
        for j in range(nj):
            acc = acc + jnp.dot(
                x_ref[pl.ds(j * blk * d * 8 // 128, blk * d * 8 // 128), :],
                w_ref[pl.ds(j * 128, 128), :],
                preferred_element_type=jnp.float32,
            )
        o_ref[...] = acc

    prows_blk = blk * _FP * d // 128
    return pl.pallas_call(
        mm_k,
        grid=(bsz // blk,),
        in_specs=[
            pl.BlockSpec((prows_blk, 128), lambda i: (i, 0)),
            pl.BlockSpec((nj * 128, d), lambda i: (0, 0)),
            pl.BlockSpec((1, d), lambda i: (0, 0)),
        ],
        out_specs=pl.BlockSpec((blk, d), lambda i: (i, 0)),
        out_shape=jax.ShapeDtypeStruct((bsz, d), jnp.float32),
    )


def kernel(inputs, table, W, b):
    bsz, f = inputs.shape
    d = table.shape[1]
    blk = 512
    nj = _FP // 8
    num_idx = bsz * _FP

    idxp = jnp.concatenate(
        [inputs.astype(jnp.int32),
         jnp.zeros((bsz, _FP - f), jnp.int32)], axis=1)
    g_cnt = blk // _LANE
    idxr = idxp.reshape(bsz // blk, g_cnt, _LANE, nj, 8).transpose(0, 3, 1, 4, 2)

    gather_fn, nw, n_stream = _build_gather(num_idx, d)
    idx3 = idxr.reshape(nw, n_stream, _LANE)
    x = gather_fn(idx3, table)

    wp = jnp.concatenate(
        [W, jnp.zeros(((_FP - f) * d, d), jnp.float32)], axis=0)
    mm = _build_matmul(bsz, d, blk, nj)
    return mm(x, wp, b.reshape(1, d))


# speedup vs baseline: 1.7697x; 1.7644x over previous
import functools

import jax
import jax.numpy as jnp
from jax import lax
from jax.experimental import pallas as pl
from jax.experimental.pallas import tpu as pltpu
from jax.experimental.pallas import tpu_sc as plsc

_LANE = 128
_SUP = 13


@functools.lru_cache(maxsize=None)
def _build_gather(num_idx, d):
    info = plsc.get_sparse_core_info()
    nc, ns = info.num_cores, info.num_subcores
    nw = nc * ns
    per_w = num_idx // nw
    assert per_w * nw == num_idx
    n_stream = per_w // _LANE
    assert n_stream * _LANE == per_w
    n_sup = n_stream // _SUP
    assert n_sup * _SUP == n_stream

    mesh = plsc.VectorSubcoreMesh(core_axis_name="c", subcore_axis_name="s")

    @functools.partial(
        pl.kernel,
        mesh=mesh,
        compiler_params=pltpu.CompilerParams(use_tc_tiling_on_sc=False),
        out_type=jax.ShapeDtypeStruct((num_idx, d), jnp.float32),
        scratch_types=(
            [pltpu.VMEM((n_stream, _LANE), jnp.int32)]
            + [pltpu.VMEM((_SUP * _LANE, d), jnp.float32)] * 2
            + [pltpu.SemaphoreType.DMA] * 4
        ),
    )
    def gather_k(idx_hbm, table_hbm, out_hbm, idx_v, buf0, buf1, g0, g1, o0, o1):
        gbufs = (buf0, buf1)
        gsems = (g0, g1)
        osems = (o0, o1)
        wid = lax.axis_index("s") * nc + lax.axis_index("c")
        pltpu.sync_copy(idx_hbm.at[wid], idx_v)

        def stream_copy(t, a):
            return pltpu.make_async_copy(
                table_hbm.at[idx_v.at[t * _SUP + a]],
                gbufs[t % 2].at[pl.ds(a * _LANE, _LANE)],
                gsems[t % 2],
            )

        def out_copy(t):
            return pltpu.make_async_copy(
                gbufs[t % 2],
                out_hbm.at[pl.ds((wid * n_sup + t) * _SUP * _LANE, _SUP * _LANE)],
                osems[t % 2],
            )

        def fire(t):
            for a in range(_SUP):
                stream_copy(t, a).start()

        fire(0)
        for t in range(n_sup):
            for a in range(_SUP):
                stream_copy(t, a).wait()
            out_copy(t).start()
            if t + 1 < n_sup:
                if t >= 1:
                    out_copy(t - 1).wait()
                fire(t + 1)
        out_copy(n_sup - 2).wait()
        out_copy(n_sup - 1).wait()

    return gather_k, nw, n_stream


@functools.lru_cache(maxsize=None)
def _build_matmul(bsz, k, d, blk):
    def mm_k(x_ref, w_ref, b_ref, o_ref):
        o_ref[...] = jnp.broadcast_to(b_ref[...], (blk, d)) + jnp.dot(
            x_ref[...], w_ref[...], preferred_element_type=jnp.float32)

    return pl.pallas_call(
        mm_k,
        grid=(bsz // blk,),
        in_specs=[
            pl.BlockSpec((blk, k), lambda i: (i, 0)),
            pl.BlockSpec((k, d), lambda i: (0, 0)),
            pl.BlockSpec((1, d), lambda i: (0, 0)),
        ],
        out_specs=pl.BlockSpec((blk, d), lambda i: (i, 0)),
        out_shape=jax.ShapeDtypeStruct((bsz, d), jnp.float32),
    )


def kernel(inputs, table, W, b):
    bsz, f = inputs.shape
    d = table.shape[1]
    num_idx = bsz * f

    gather_fn, nw, n_stream = _build_gather(num_idx, d)
    idx3 = inputs.astype(jnp.int32).reshape(nw, n_stream, _LANE)
    x = gather_fn(idx3, table)

    mm = _build_matmul(bsz, f * d, d, 512)
    return mm(x.reshape(bsz, f * d), W, b.reshape(1, d))
